# baseline (speedup 1.0000x reference)
import functools

import jax
import jax.numpy as jnp
from jax import lax
from jax.experimental import pallas as pl
from jax.experimental.pallas import tpu as pltpu
from jax.experimental.pallas import tpu_sc as plsc

_NUM_CORES = 2
_NUM_SUBCORES = 16
_NUM_WORKERS = _NUM_CORES * _NUM_SUBCORES

_SC_FRACTION = 0.5
_SC_CHUNK = 16
_TC_BLOCK = 1024


def _pipeline(read_fn, write_fns, n_chunks):
    read_fn(0).start()
    if n_chunks > 1:
        read_fn(1).start()
    for t in range(n_chunks):
        read_fn(t).wait()
        for w in write_fns(t):
            w.start()
        if t + 2 < n_chunks:
            for w in write_fns(t):
                w.wait()
            read_fn(t + 2).start()
    for t in (n_chunks - 2, n_chunks - 1):
        if t >= 0 and t + 2 >= n_chunks:
            for w in write_fns(t):
                w.wait()


@functools.cache
def _build_sc(batch, seq_len, d_model, dtype, sc_rows):
    rows_per_w = sc_rows // _NUM_WORKERS
    n_chunks = rows_per_w // _SC_CHUNK

    mesh = plsc.VectorSubcoreMesh(
        core_axis_name="c", subcore_axis_name="s", num_cores=_NUM_CORES
    )

    @functools.partial(
        pl.kernel,
        out_type=jax.ShapeDtypeStruct((batch, seq_len, d_model), dtype),
        mesh=mesh,
        scratch_types=[
            pltpu.VMEM((_SC_CHUNK, d_model), dtype),
            pltpu.VMEM((_SC_CHUNK, d_model), dtype),
            pltpu.SemaphoreType.DMA,
            pltpu.SemaphoreType.DMA,
            pltpu.SemaphoreType.DMA,
            pltpu.SemaphoreType.DMA,
        ],
    )
    def broadcast_rows(table_hbm, out_hbm, buf0, buf1, rsem0, rsem1, wsem0, wsem1):
        bufs = (buf0, buf1)
        rsems = (rsem0, rsem1)
        wsems = (wsem0, wsem1)
        wid = lax.axis_index("s") * _NUM_CORES + lax.axis_index("c")
        base = wid * rows_per_w

        def read(t):
            return pltpu.make_async_copy(
                table_hbm.at[pl.ds(base + t * _SC_CHUNK, _SC_CHUNK)],
                bufs[t % 2],
                rsems[t % 2],
            )

        def writes(t):
            return [
                pltpu.make_async_copy(
                    bufs[t % 2],
                    out_hbm.at[b].at[pl.ds(base + t * _SC_CHUNK, _SC_CHUNK)],
                    wsems[t % 2],
                )
                for b in range(batch)
            ]

        _pipeline(read, writes, n_chunks)

    return broadcast_rows


@functools.cache
def _build_tc(batch, seq_len, d_model, dtype, sc_rows):
    tc_rows = seq_len - sc_rows
    n_chunks = tc_rows // _TC_BLOCK

    def tc_body(table_ref, partial_ref, out_ref, buf0, buf1, rsem0, rsem1,
                wsem0, wsem1):
        del partial_ref
        bufs = (buf0, buf1)
        rsems = (rsem0, rsem1)
        wsems = (wsem0, wsem1)

        def read(t):
            return pltpu.make_async_copy(
                table_ref.at[pl.ds(sc_rows + t * _TC_BLOCK, _TC_BLOCK)],
                bufs[t % 2],
                rsems[t % 2],
            )

        def writes(t):
            return [
                pltpu.make_async_copy(
                    bufs[t % 2],
                    out_ref.at[b].at[pl.ds(sc_rows + t * _TC_BLOCK, _TC_BLOCK)],
                    wsems[t % 2],
                )
                for b in range(batch)
            ]

        _pipeline(read, writes, n_chunks)

    return pl.pallas_call(
        tc_body,
        in_specs=[
            pl.BlockSpec(memory_space=pl.ANY),
            pl.BlockSpec(memory_space=pl.ANY),
        ],
        out_specs=pl.BlockSpec(memory_space=pl.ANY),
        out_shape=jax.ShapeDtypeStruct((batch, seq_len, d_model), dtype),
        scratch_shapes=[
            pltpu.VMEM((_TC_BLOCK, d_model), dtype),
            pltpu.VMEM((_TC_BLOCK, d_model), dtype),
            pltpu.SemaphoreType.DMA,
            pltpu.SemaphoreType.DMA,
            pltpu.SemaphoreType.DMA,
            pltpu.SemaphoreType.DMA,
        ],
        input_output_aliases={1: 0},
    )


def kernel(inputs, pos_embedding):
    batch, seq_len = inputs.shape
    _, d_model = pos_embedding.shape
    sc_rows = int(seq_len * _SC_FRACTION)
    sc_rows -= sc_rows % max(_NUM_WORKERS * _SC_CHUNK, _TC_BLOCK)
    dtype = pos_embedding.dtype
    partial_out = _build_sc(batch, seq_len, d_model, dtype, sc_rows)(pos_embedding)
    return _build_tc(batch, seq_len, d_model, dtype, sc_rows)(
        pos_embedding, partial_out
    )


# speedup vs baseline: 1.0343x; 1.0343x over previous
import functools

import jax
import jax.numpy as jnp
from jax import lax
from jax.experimental import pallas as pl
from jax.experimental.pallas import tpu as pltpu
from jax.experimental.pallas import tpu_sc as plsc

_NUM_CORES = 2
_NUM_SUBCORES = 16
_NUM_WORKERS = _NUM_CORES * _NUM_SUBCORES

_SC_FRACTION = 0.5
_SC_CHUNK = 32
_TC_BLOCK = 1024


def _pipeline(read_fn, write_fns, n_chunks):
    read_fn(0).start()
    if n_chunks > 1:
        read_fn(1).start()
    for t in range(n_chunks):
        read_fn(t).wait()
        for w in write_fns(t):
            w.start()
        if t + 2 < n_chunks:
            for w in write_fns(t):
                w.wait()
            read_fn(t + 2).start()
    for t in (n_chunks - 2, n_chunks - 1):
        if t >= 0 and t + 2 >= n_chunks:
            for w in write_fns(t):
                w.wait()


@functools.cache
def _build_sc(batch, seq_len, d_model, dtype, sc_rows):
    rows_per_w = sc_rows // _NUM_WORKERS
    n_chunks = rows_per_w // _SC_CHUNK

    mesh = plsc.VectorSubcoreMesh(
        core_axis_name="c", subcore_axis_name="s", num_cores=_NUM_CORES
    )

    @functools.partial(
        pl.kernel,
        out_type=jax.ShapeDtypeStruct((batch, seq_len, d_model), dtype),
        mesh=mesh,
        scratch_types=[
            pltpu.VMEM((_SC_CHUNK, d_model), dtype),
            pltpu.VMEM((_SC_CHUNK, d_model), dtype),
            pltpu.SemaphoreType.DMA,
            pltpu.SemaphoreType.DMA,
            pltpu.SemaphoreType.DMA,
            pltpu.SemaphoreType.DMA,
        ],
    )
    def broadcast_rows(table_hbm, out_hbm, buf0, buf1, rsem0, rsem1, wsem0, wsem1):
        bufs = (buf0, buf1)
        rsems = (rsem0, rsem1)
        wsems = (wsem0, wsem1)
        wid = lax.axis_index("s") * _NUM_CORES + lax.axis_index("c")
        base = wid * rows_per_w

        def read(t):
            return pltpu.make_async_copy(
                table_hbm.at[pl.ds(base + t * _SC_CHUNK, _SC_CHUNK)],
                bufs[t % 2],
                rsems[t % 2],
            )

        def writes(t):
            return [
                pltpu.make_async_copy(
                    bufs[t % 2],
                    out_hbm.at[b].at[pl.ds(base + t * _SC_CHUNK, _SC_CHUNK)],
                    wsems[t % 2],
                )
                for b in range(batch)
            ]

        _pipeline(read, writes, n_chunks)

    return broadcast_rows


@functools.cache
def _build_tc(batch, seq_len, d_model, dtype, sc_rows):
    tc_rows = seq_len - sc_rows
    n_chunks = tc_rows // _TC_BLOCK

    def tc_body(table_ref, partial_ref, out_ref, buf0, buf1, rsem0, rsem1,
                wsem0, wsem1):
        del partial_ref
        bufs = (buf0, buf1)
        rsems = (rsem0, rsem1)
        wsems = (wsem0, wsem1)

        def read(t):
            return pltpu.make_async_copy(
                table_ref.at[pl.ds(sc_rows + t * _TC_BLOCK, _TC_BLOCK)],
                bufs[t % 2],
                rsems[t % 2],
            )

        def writes(t):
            return [
                pltpu.make_async_copy(
                    bufs[t % 2],
                    out_ref.at[b].at[pl.ds(sc_rows + t * _TC_BLOCK, _TC_BLOCK)],
                    wsems[t % 2],
                )
                for b in range(batch)
            ]

        _pipeline(read, writes, n_chunks)

    return pl.pallas_call(
        tc_body,
        in_specs=[
            pl.BlockSpec(memory_space=pl.ANY),
            pl.BlockSpec(memory_space=pl.ANY),
        ],
        out_specs=pl.BlockSpec(memory_space=pl.ANY),
        out_shape=jax.ShapeDtypeStruct((batch, seq_len, d_model), dtype),
        scratch_shapes=[
            pltpu.VMEM((_TC_BLOCK, d_model), dtype),
            pltpu.VMEM((_TC_BLOCK, d_model), dtype),
            pltpu.SemaphoreType.DMA,
            pltpu.SemaphoreType.DMA,
            pltpu.SemaphoreType.DMA,
            pltpu.SemaphoreType.DMA,
        ],
        input_output_aliases={1: 0},
    )


def kernel(inputs, pos_embedding):
    batch, seq_len = inputs.shape
    _, d_model = pos_embedding.shape
    sc_rows = int(seq_len * _SC_FRACTION)
    sc_rows -= sc_rows % max(_NUM_WORKERS * _SC_CHUNK, _TC_BLOCK)
    dtype = pos_embedding.dtype
    partial_out = _build_sc(batch, seq_len, d_model, dtype, sc_rows)(pos_embedding)
    return _build_tc(batch, seq_len, d_model, dtype, sc_rows)(
        pos_embedding, partial_out
    )
